# baseline (speedup 1.0000x reference)
import jax
import jax.numpy as jnp
from jax import lax
from jax.experimental import pallas as pl
from jax.experimental.pallas import tpu as pltpu
from jax.experimental.pallas import tpu_sc as plsc

VOCAB = 100000
HID = 768
B = 4
S = 4096
N = B * S
EPS = 1e-12

NC = 2
NS = 16
NW = NC * NS
K = 2
NSL = N // K
TPW = NSL // NW
C = 32
NCH = TPW // C
HC = HID // 16

TB = 512


def _sc_body(ids, pids, wtab, ptab, out,
             idx_v, pidx_v, xw0, xp0, xw1, xp1, obuf,
             semg0, semg1, semo):
    wid = lax.axis_index("s") * NC + lax.axis_index("c")
    base = pl.multiple_of(wid * TPW, TPW)
    pltpu.sync_copy(ids.at[pl.ds(base, TPW)], idx_v)
    pltpu.sync_copy(pids.at[pl.ds(base, TPW)], pidx_v)

    bufs = ((xw0, xp0, semg0), (xw1, xp1, semg1))

    def issue_gathers(g, xw, xp, semg):
        off = pl.multiple_of(g * C, C)
        pltpu.async_copy(wtab.at[idx_v.at[pl.ds(off, C)]], xw, semg)
        pltpu.async_copy(ptab.at[pidx_v.at[pl.ds(off, C)]], xp, semg)

    issue_gathers(0, xw0, xp0, semg0)

    def pair_body(i, carry):
        for par in (0, 1):
            g = i * 2 + par
            xw_c, xp_c, semg_c = bufs[par]
            xw_n, xp_n, semg_n = bufs[1 - par]

            @pl.when(g + 1 < NCH)
            def _():
                issue_gathers(g + 1, xw_n, xp_n, semg_n)

            pltpu.make_async_copy(out.at[pl.ds(0, C)], xw_c, semg_c).wait()
            pltpu.make_async_copy(out.at[pl.ds(0, C)], xp_c, semg_c).wait()

            @pl.when(g >= 1)
            def _():
                pltpu.make_async_copy(obuf, out.at[pl.ds(0, C)],
                                      semo).wait()

            off = pl.multiple_of(g * C, C)

            def sum_body(ti, carry2):
                for k in range(2):
                    t = ti * 2 + k
                    for j in range(HC):
                        sl = pl.ds(j * 16, 16)
                        obuf[t, sl] = xw_c[t, sl] + xp_c[t, sl]
                return carry2

            lax.fori_loop(0, C // 2, sum_body, 0)
            pltpu.async_copy(obuf, out.at[pl.ds(base + off, C)], semo)
        return carry

    lax.fori_loop(0, NCH // 2, pair_body, 0)
    pltpu.make_async_copy(obuf, out.at[pl.ds(0, C)], semo).wait()


def _tc_ln_body(prev_ref, x_ref, tt_ref, ttab_ref, g_ref, b_ref, o_ref):
    x = x_ref[...]
    mf = tt_ref[0, 0, :].astype(jnp.float32)
    t0 = ttab_ref[0, :]
    d = ttab_ref[1, :] - t0
    x = x + t0[None, :] + mf[:, None] * d[None, :]
    mean = jnp.mean(x, axis=-1, keepdims=True)
    xc = x - mean
    var = jnp.mean(xc * xc, axis=-1, keepdims=True)
    y = xc * lax.rsqrt(var + EPS)
    o_ref[...] = y * g_ref[0, :][None, :] + b_ref[0, :][None, :]


def kernel(input_ids, token_type_ids, turn_type_ids, word_table, type_table,
           pos_table, ln_gamma, ln_beta):
    ids = input_ids.reshape(-1)
    tts = token_type_ids.reshape(-1)
    pids = turn_type_ids.reshape(-1)

    mesh = plsc.VectorSubcoreMesh(core_axis_name="c", subcore_axis_name="s")
    sc = pl.kernel(
        _sc_body,
        out_type=jax.ShapeDtypeStruct((NSL, HID), jnp.float32),
        mesh=mesh,
        scratch_types=[
            pltpu.VMEM((TPW,), jnp.int32),
            pltpu.VMEM((TPW,), jnp.int32),
            pltpu.VMEM((C, HID), jnp.float32),
            pltpu.VMEM((C, HID), jnp.float32),
            pltpu.VMEM((C, HID), jnp.float32),
            pltpu.VMEM((C, HID), jnp.float32),
            pltpu.VMEM((C, HID), jnp.float32),
            pltpu.SemaphoreType.DMA,
            pltpu.SemaphoreType.DMA,
            pltpu.SemaphoreType.DMA,
        ],
    )
    xs = [sc(ids[i * NSL:(i + 1) * NSL], pids[i * NSL:(i + 1) * NSL],
             word_table, pos_table) for i in range(K)]

    nbs = NSL // TB
    out = jnp.zeros((N, HID), jnp.float32)
    for i in range(K):
        out = pl.pallas_call(
            _tc_ln_body,
            out_shape=jax.ShapeDtypeStruct((N, HID), jnp.float32),
            grid=(nbs,),
            in_specs=[
                pl.BlockSpec(memory_space=pl.ANY),
                pl.BlockSpec((TB, HID), lambda g: (g, 0)),
                pl.BlockSpec((1, 1, TB), lambda g: (g, 0, 0)),
                pl.BlockSpec((2, HID), lambda g: (0, 0)),
                pl.BlockSpec((1, HID), lambda g: (0, 0)),
                pl.BlockSpec((1, HID), lambda g: (0, 0)),
            ],
            out_specs=pl.BlockSpec((TB, HID),
                                   lambda g, _i=i: (_i * nbs + g, 0)),
            input_output_aliases={0: 0},
        )(out, xs[i],
          tts[i * NSL:(i + 1) * NSL].reshape(nbs, 1, TB), type_table,
          ln_gamma.reshape(1, HID), ln_beta.reshape(1, HID))
    return out.reshape(B, S, HID)


# speedup vs baseline: 1.1601x; 1.1601x over previous
import jax
import jax.numpy as jnp
from jax import lax
from jax.experimental import pallas as pl
from jax.experimental.pallas import tpu as pltpu
from jax.experimental.pallas import tpu_sc as plsc

VOCAB = 100000
HID = 768
B = 4
S = 4096
N = B * S
EPS = 1e-12

NC = 2
NS = 16
NW = NC * NS
TPW = N // NW
C = 32
NCH = TPW // C
HC = HID // 16

TB = 1024


def _sc_body(ids, pids, wtab, ptab, out,
             idx_v, pidx_v, xw0, xp0, xw1, xp1, obuf,
             semg0, semg1, semo):
    wid = lax.axis_index("s") * NC + lax.axis_index("c")
    base = pl.multiple_of(wid * TPW, TPW)
    pltpu.sync_copy(ids.at[pl.ds(base, TPW)], idx_v)
    pltpu.sync_copy(pids.at[pl.ds(base, TPW)], pidx_v)

    bufs = ((xw0, xp0, semg0), (xw1, xp1, semg1))

    def issue_gathers(g, xw, xp, semg):
        off = pl.multiple_of(g * C, C)
        pltpu.async_copy(wtab.at[idx_v.at[pl.ds(off, C)]], xw, semg)
        pltpu.async_copy(ptab.at[pidx_v.at[pl.ds(off, C)]], xp, semg)

    issue_gathers(0, xw0, xp0, semg0)

    def pair_body(i, carry):
        for par in (0, 1):
            g = i * 2 + par
            xw_c, xp_c, semg_c = bufs[par]
            xw_n, xp_n, semg_n = bufs[1 - par]

            @pl.when(g + 1 < NCH)
            def _():
                issue_gathers(g + 1, xw_n, xp_n, semg_n)

            pltpu.make_async_copy(out.at[pl.ds(0, C)], xw_c, semg_c).wait()
            pltpu.make_async_copy(out.at[pl.ds(0, C)], xp_c, semg_c).wait()

            @pl.when(g >= 1)
            def _():
                pltpu.make_async_copy(obuf, out.at[pl.ds(0, C)],
                                      semo).wait()

            off = pl.multiple_of(g * C, C)

            def sum_body(ti, carry2):
                for k in range(2):
                    t = ti * 2 + k
                    for j in range(HC):
                        sl = pl.ds(j * 16, 16)
                        obuf[t, sl] = xw_c[t, sl] + xp_c[t, sl]
                return carry2

            lax.fori_loop(0, C // 2, sum_body, 0)
            pltpu.async_copy(obuf, out.at[pl.ds(base + off, C)], semo)
        return carry

    lax.fori_loop(0, NCH // 2, pair_body, 0)
    pltpu.make_async_copy(obuf, out.at[pl.ds(0, C)], semo).wait()


def _tc_ln_body(x_ref, tt_ref, ttab_ref, g_ref, b_ref, o_ref):
    x = x_ref[...]
    mf = tt_ref[0, 0, :].astype(jnp.float32)
    t0 = ttab_ref[0, :]
    d = ttab_ref[1, :] - t0
    x = x + t0[None, :] + mf[:, None] * d[None, :]
    mean = jnp.mean(x, axis=-1, keepdims=True)
    xc = x - mean
    var = jnp.mean(xc * xc, axis=-1, keepdims=True)
    y = xc * lax.rsqrt(var + EPS)
    o_ref[...] = y * g_ref[0, :][None, :] + b_ref[0, :][None, :]


def kernel(input_ids, token_type_ids, turn_type_ids, word_table, type_table,
           pos_table, ln_gamma, ln_beta):
    ids = input_ids.reshape(-1)
    tts = token_type_ids.reshape(-1)
    pids = turn_type_ids.reshape(-1)

    mesh = plsc.VectorSubcoreMesh(core_axis_name="c", subcore_axis_name="s")
    sc = pl.kernel(
        _sc_body,
        out_type=jax.ShapeDtypeStruct((N, HID), jnp.float32),
        mesh=mesh,
        scratch_types=[
            pltpu.VMEM((TPW,), jnp.int32),
            pltpu.VMEM((TPW,), jnp.int32),
            pltpu.VMEM((C, HID), jnp.float32),
            pltpu.VMEM((C, HID), jnp.float32),
            pltpu.VMEM((C, HID), jnp.float32),
            pltpu.VMEM((C, HID), jnp.float32),
            pltpu.VMEM((C, HID), jnp.float32),
            pltpu.SemaphoreType.DMA,
            pltpu.SemaphoreType.DMA,
            pltpu.SemaphoreType.DMA,
        ],
    )
    x = sc(ids, pids, word_table, pos_table)

    nb = N // TB
    out = pl.pallas_call(
        _tc_ln_body,
        out_shape=jax.ShapeDtypeStruct((N, HID), jnp.float32),
        grid=(nb,),
        in_specs=[
            pl.BlockSpec((TB, HID), lambda i: (i, 0)),
            pl.BlockSpec((1, 1, TB), lambda i: (i, 0, 0)),
            pl.BlockSpec((2, HID), lambda i: (0, 0)),
            pl.BlockSpec((1, HID), lambda i: (0, 0)),
            pl.BlockSpec((1, HID), lambda i: (0, 0)),
        ],
        out_specs=pl.BlockSpec((TB, HID), lambda i: (i, 0)),
    )(x, tts.reshape(nb, 1, TB), type_table, ln_gamma.reshape(1, HID),
      ln_beta.reshape(1, HID))
    return out.reshape(B, S, HID)


# speedup vs baseline: 1.1900x; 1.0258x over previous
import jax
import jax.numpy as jnp
from jax import lax
from jax.experimental import pallas as pl
from jax.experimental.pallas import tpu as pltpu
from jax.experimental.pallas import tpu_sc as plsc

VOCAB = 100000
HID = 768
B = 4
S = 4096
N = B * S
EPS = 1e-12

NC = 2
NS = 16
NW = NC * NS
TPW = N // NW
C = 32
NCH = TPW // C
HC = HID // 16

TB = 2048


def _sc_body(ids, pids, wtab, ptab, out,
             idx_v, pidx_v, xw0, xp0, xw1, xp1, obuf,
             semg0, semg1, semo):
    wid = lax.axis_index("s") * NC + lax.axis_index("c")
    base = pl.multiple_of(wid * TPW, TPW)
    pltpu.sync_copy(ids.at[pl.ds(base, TPW)], idx_v)
    pltpu.sync_copy(pids.at[pl.ds(base, TPW)], pidx_v)

    bufs = ((xw0, xp0, semg0), (xw1, xp1, semg1))

    def issue_gathers(g, xw, xp, semg):
        off = pl.multiple_of(g * C, C)
        pltpu.async_copy(wtab.at[idx_v.at[pl.ds(off, C)]], xw, semg)
        pltpu.async_copy(ptab.at[pidx_v.at[pl.ds(off, C)]], xp, semg)

    issue_gathers(0, xw0, xp0, semg0)

    def pair_body(i, carry):
        for par in (0, 1):
            g = i * 2 + par
            xw_c, xp_c, semg_c = bufs[par]
            xw_n, xp_n, semg_n = bufs[1 - par]

            @pl.when(g + 1 < NCH)
            def _():
                issue_gathers(g + 1, xw_n, xp_n, semg_n)

            pltpu.make_async_copy(out.at[pl.ds(0, C)], xw_c, semg_c).wait()
            pltpu.make_async_copy(out.at[pl.ds(0, C)], xp_c, semg_c).wait()

            @pl.when(g >= 1)
            def _():
                pltpu.make_async_copy(obuf, out.at[pl.ds(0, C)],
                                      semo).wait()

            off = pl.multiple_of(g * C, C)

            def sum_body(ti, carry2):
                for k in range(2):
                    t = ti * 2 + k
                    for j in range(HC):
                        sl = pl.ds(j * 16, 16)
                        obuf[t, sl] = xw_c[t, sl] + xp_c[t, sl]
                return carry2

            lax.fori_loop(0, C // 2, sum_body, 0)
            pltpu.async_copy(obuf, out.at[pl.ds(base + off, C)], semo)
        return carry

    lax.fori_loop(0, NCH // 2, pair_body, 0)
    pltpu.make_async_copy(obuf, out.at[pl.ds(0, C)], semo).wait()


def _tc_ln_body(x_ref, tt_ref, ttab_ref, g_ref, b_ref, o_ref):
    x = x_ref[...]
    mf = tt_ref[0, 0, :].astype(jnp.float32)
    t0 = ttab_ref[0, :]
    d = ttab_ref[1, :] - t0
    x = x + t0[None, :] + mf[:, None] * d[None, :]
    mean = jnp.mean(x, axis=-1, keepdims=True)
    xc = x - mean
    var = jnp.mean(xc * xc, axis=-1, keepdims=True)
    y = xc * lax.rsqrt(var + EPS)
    o_ref[...] = y * g_ref[0, :][None, :] + b_ref[0, :][None, :]


def kernel(input_ids, token_type_ids, turn_type_ids, word_table, type_table,
           pos_table, ln_gamma, ln_beta):
    ids = input_ids.reshape(-1)
    tts = token_type_ids.reshape(-1)
    pids = turn_type_ids.reshape(-1)

    mesh = plsc.VectorSubcoreMesh(core_axis_name="c", subcore_axis_name="s")
    sc = pl.kernel(
        _sc_body,
        out_type=jax.ShapeDtypeStruct((N, HID), jnp.float32),
        mesh=mesh,
        scratch_types=[
            pltpu.VMEM((TPW,), jnp.int32),
            pltpu.VMEM((TPW,), jnp.int32),
            pltpu.VMEM((C, HID), jnp.float32),
            pltpu.VMEM((C, HID), jnp.float32),
            pltpu.VMEM((C, HID), jnp.float32),
            pltpu.VMEM((C, HID), jnp.float32),
            pltpu.VMEM((C, HID), jnp.float32),
            pltpu.SemaphoreType.DMA,
            pltpu.SemaphoreType.DMA,
            pltpu.SemaphoreType.DMA,
        ],
    )
    x = sc(ids, pids, word_table, pos_table)

    nb = N // TB
    out = pl.pallas_call(
        _tc_ln_body,
        out_shape=jax.ShapeDtypeStruct((N, HID), jnp.float32),
        grid=(nb,),
        in_specs=[
            pl.BlockSpec((TB, HID), lambda i: (i, 0)),
            pl.BlockSpec((1, 1, TB), lambda i: (i, 0, 0)),
            pl.BlockSpec((2, HID), lambda i: (0, 0)),
            pl.BlockSpec((1, HID), lambda i: (0, 0)),
            pl.BlockSpec((1, HID), lambda i: (0, 0)),
        ],
        out_specs=pl.BlockSpec((TB, HID), lambda i: (i, 0)),
    )(x, tts.reshape(nb, 1, TB), type_table, ln_gamma.reshape(1, HID),
      ln_beta.reshape(1, HID))
    return out.reshape(B, S, HID)
